# baseline (speedup 1.0000x reference)
import functools

import jax
import jax.numpy as jnp
from jax import lax
from jax.experimental import pallas as pl
from jax.experimental.pallas import tpu as pltpu
from jax.experimental.pallas import tpu_sc as plsc

B = 16384
D = 128
NC = 2
NS = 16
L = 16
NW = NC * NS
BPW = B // NW
C = 64
NG = BPW // C
NBUF = 2
U = 2

_BITREV4 = [int(f"{e:04b}"[::-1], 2) for e in range(L)]
_HS = (8, 4, 2, 1)


def _bpr_body(uid_hbm, pid_hbm, nid_hbm, utab_hbm, itab_hbm,
              pos_hbm, neg_hbm,
              idx_u, idx_p, idx_n, u_rows, p_rows, n_rows,
              part_p, part_n, pos_v, neg_v,
              sem_i, sem_0, sem_1):
    cid = lax.axis_index("c")
    sid = lax.axis_index("s")
    wid = sid * NC + cid
    base = wid * BPW

    sems = (sem_0, sem_1)

    def start_u(g, b):
        return pltpu.async_copy(utab_hbm.at[idx_u.at[pl.ds(g * C, C)]],
                                u_rows.at[b], sems[b])

    def start_pn(g, b):
        return (
            pltpu.async_copy(itab_hbm.at[idx_p.at[pl.ds(g * C, C)]],
                             p_rows.at[b], sems[b]),
            pltpu.async_copy(itab_hbm.at[idx_n.at[pl.ds(g * C, C)]],
                             n_rows.at[b], sems[b]),
        )

    cp_u = pltpu.async_copy(uid_hbm.at[pl.ds(base, BPW)], idx_u, sem_i)
    cp_p = pltpu.async_copy(pid_hbm.at[pl.ds(base, BPW)], idx_p, sem_i)
    cp_n = pltpu.async_copy(nid_hbm.at[pl.ds(base, BPW)], idx_n, sem_i)
    cp_u.wait()
    for g in range(NBUF):
        start_u(g, g)
    cp_p.wait()
    cp_n.wait()
    for g in range(NBUF):
        start_pn(g, g)

    lane = lax.iota(jnp.int32, L)
    dnums = lax.GatherDimensionNumbers(
        offset_dims=(), collapsed_slice_dims=(0,), start_index_map=(0,))

    def take16(v, idx):
        return lax.gather(v, idx[:, None], dnums, slice_sizes=(1,),
                          mode=lax.GatherScatterMode.PROMISE_IN_BOUNDS)

    def combine(a, b, h):
        clear = (lane & h) == 0
        x = jnp.where(clear, a, b)
        y = jnp.where(clear, b, a)
        return x + take16(y, lane ^ h)

    def compute(g, b):
        def rows_body(i, carry):
            for k in range(U):
                row = i * U + k
                u = [u_rows[b, row, pl.ds(j * L, L)] for j in range(D // L)]
                p = [p_rows[b, row, pl.ds(j * L, L)] for j in range(D // L)]
                nn = [n_rows[b, row, pl.ds(j * L, L)] for j in range(D // L)]
                up = [a * c for a, c in zip(u, p)]
                un = [a * c for a, c in zip(u, nn)]
                accp = (((up[0] + up[1]) + (up[2] + up[3]))
                        + ((up[4] + up[5]) + (up[6] + up[7])))
                accn = (((un[0] + un[1]) + (un[2] + un[3]))
                        + ((un[4] + un[5]) + (un[6] + un[7])))
                part_p[pl.ds(row * L, L)] = accp
                part_n[pl.ds(row * L, L)] = accn
            return carry

        lax.fori_loop(0, C // U, rows_body, 0)

        def grp_body(grp, carry):
            base_row = grp * L
            for part, out in ((part_p, pos_v), (part_n, neg_v)):
                stk = []
                for e in _BITREV4:
                    vec = part[pl.ds((base_row + e) * L, L)]
                    lvl = 0
                    while stk and stk[-1][0] == lvl:
                        _, left = stk.pop()
                        vec = combine(left, vec, _HS[lvl])
                        lvl += 1
                    stk.append((lvl, vec))
                out[pl.ds(g * C + base_row, L)] = stk[0][1]
            return carry

        lax.fori_loop(0, C // L, grp_body, 0)

    def outer(i, carry):
        for b in range(NBUF):
            g = i * NBUF + b
            pltpu.make_async_copy(utab_hbm.at[pl.ds(0, C)],
                                  u_rows.at[b], sems[b]).wait()
            pltpu.make_async_copy(itab_hbm.at[pl.ds(0, C)],
                                  p_rows.at[b], sems[b]).wait()
            pltpu.make_async_copy(itab_hbm.at[pl.ds(0, C)],
                                  n_rows.at[b], sems[b]).wait()
            compute(g, b)

            @pl.when(i < NG // NBUF - 1)
            def _prefetch():
                g2 = g + NBUF
                start_u(g2, b)
                start_pn(g2, b)
        return carry

    lax.fori_loop(0, NG // NBUF, outer, 0)

    cp_op = pltpu.async_copy(pos_v, pos_hbm.at[pl.ds(base, BPW)], sem_i)
    cp_on = pltpu.async_copy(neg_v, neg_hbm.at[pl.ds(base, BPW)], sem_i)
    cp_op.wait()
    cp_on.wait()


@jax.jit
def _bpr(uid, pid, nid, user_table, item_table):
    mesh = plsc.VectorSubcoreMesh(core_axis_name="c", subcore_axis_name="s")
    run = functools.partial(
        pl.kernel,
        out_type=(
            jax.ShapeDtypeStruct((B,), jnp.float32),
            jax.ShapeDtypeStruct((B,), jnp.float32),
        ),
        mesh=mesh,
        scratch_types=(
            pltpu.VMEM((BPW,), jnp.int32),
            pltpu.VMEM((BPW,), jnp.int32),
            pltpu.VMEM((BPW,), jnp.int32),
            pltpu.VMEM((NBUF, C, D), jnp.float32),
            pltpu.VMEM((NBUF, C, D), jnp.float32),
            pltpu.VMEM((NBUF, C, D), jnp.float32),
            pltpu.VMEM((C * L,), jnp.float32),
            pltpu.VMEM((C * L,), jnp.float32),
            pltpu.VMEM((BPW,), jnp.float32),
            pltpu.VMEM((BPW,), jnp.float32),
            pltpu.SemaphoreType.DMA,
            pltpu.SemaphoreType.DMA,
            pltpu.SemaphoreType.DMA,
        ),
    )(_bpr_body)
    return run(uid, pid, nid, user_table, item_table)


def kernel(userId, itemId, neg_itemId, user_table, item_table):
    return _bpr(userId, itemId, neg_itemId, user_table, item_table)


# speedup vs baseline: 1.0227x; 1.0227x over previous
import functools

import jax
import jax.numpy as jnp
from jax import lax
from jax.experimental import pallas as pl
from jax.experimental.pallas import tpu as pltpu
from jax.experimental.pallas import tpu_sc as plsc

B = 16384
D = 128
NC = 2
NS = 16
L = 16
NW = NC * NS
BPW = B // NW
C = 64
NG = BPW // C
NBUF = 2
U = 2

_BITREV4 = [int(f"{e:04b}"[::-1], 2) for e in range(L)]
_HS = (8, 4, 2, 1)


def _bpr_body(uid_hbm, pid_hbm, nid_hbm, utab_hbm, itab_hbm,
              pos_hbm, neg_hbm,
              idx_u, idx_p, idx_n, u_rows, p_rows, n_rows,
              part_p, part_n, pos_v, neg_v,
              sem_i, sem_0, sem_1):
    cid = lax.axis_index("c")
    sid = lax.axis_index("s")
    wid = sid * NC + cid
    base = wid * BPW

    sems = (sem_0, sem_1)

    def start_u(g, b):
        return pltpu.async_copy(utab_hbm.at[idx_u.at[pl.ds(g * C, C)]],
                                u_rows.at[b], sems[b])

    def start_pn(g, b):
        return (
            pltpu.async_copy(itab_hbm.at[idx_p.at[pl.ds(g * C, C)]],
                             p_rows.at[b], sems[b]),
            pltpu.async_copy(itab_hbm.at[idx_n.at[pl.ds(g * C, C)]],
                             n_rows.at[b], sems[b]),
        )

    cp_u = pltpu.async_copy(uid_hbm.at[pl.ds(base, BPW)], idx_u, sem_i)
    cp_p = pltpu.async_copy(pid_hbm.at[pl.ds(base, BPW)], idx_p, sem_i)
    cp_n = pltpu.async_copy(nid_hbm.at[pl.ds(base, BPW)], idx_n, sem_i)
    cp_u.wait()
    for g in range(NBUF):
        start_u(g, g)
    cp_p.wait()
    cp_n.wait()
    for g in range(NBUF):
        start_pn(g, g)

    lane = lax.iota(jnp.int32, L)
    dnums = lax.GatherDimensionNumbers(
        offset_dims=(), collapsed_slice_dims=(0,), start_index_map=(0,))

    def take16(v, idx):
        return lax.gather(v, idx[:, None], dnums, slice_sizes=(1,),
                          mode=lax.GatherScatterMode.PROMISE_IN_BOUNDS)

    def combine(a, b, h):
        clear = (lane & h) == 0
        x = jnp.where(clear, a, b)
        y = jnp.where(clear, b, a)
        return x + take16(y, lane ^ h)

    def compute(g, b):
        def rows_body(i, carry):
            for k in range(U):
                row = i * U + k
                u = [u_rows[b, row, pl.ds(j * L, L)] for j in range(D // L)]
                p = [p_rows[b, row, pl.ds(j * L, L)] for j in range(D // L)]
                nn = [n_rows[b, row, pl.ds(j * L, L)] for j in range(D // L)]
                up = [a * c for a, c in zip(u, p)]
                un = [a * c for a, c in zip(u, nn)]
                accp = (((up[0] + up[1]) + (up[2] + up[3]))
                        + ((up[4] + up[5]) + (up[6] + up[7])))
                accn = (((un[0] + un[1]) + (un[2] + un[3]))
                        + ((un[4] + un[5]) + (un[6] + un[7])))
                part_p[pl.ds(row * L, L)] = accp
                part_n[pl.ds(row * L, L)] = accn
            return carry

        lax.fori_loop(0, C // U, rows_body, 0)

        def grp_body(grp, carry):
            base_row = grp * L
            for part, out in ((part_p, pos_v), (part_n, neg_v)):
                stk = []
                for e in _BITREV4:
                    vec = part[pl.ds((base_row + e) * L, L)]
                    lvl = 0
                    while stk and stk[-1][0] == lvl:
                        _, left = stk.pop()
                        vec = combine(left, vec, _HS[lvl])
                        lvl += 1
                    stk.append((lvl, vec))
                out[pl.ds(g * C + base_row, L)] = stk[0][1]
            return carry

        lax.fori_loop(0, C // L, grp_body, 0)

    def outer(g, carry):
        bt = lax.rem(g, NBUF)
        for b in range(NBUF):
            @pl.when(bt == b)
            def _wait():
                pltpu.make_async_copy(utab_hbm.at[pl.ds(0, C)],
                                      u_rows.at[b], sems[b]).wait()
                pltpu.make_async_copy(itab_hbm.at[pl.ds(0, C)],
                                      p_rows.at[b], sems[b]).wait()
                pltpu.make_async_copy(itab_hbm.at[pl.ds(0, C)],
                                      n_rows.at[b], sems[b]).wait()

        compute(g, bt)

        @pl.when(g < NG - NBUF)
        def _prefetch():
            g2 = g + NBUF
            for b in range(NBUF):
                @pl.when(bt == b)
                def _start():
                    start_u(g2, b)
                    start_pn(g2, b)
        return carry

    lax.fori_loop(0, NG, outer, 0)

    cp_op = pltpu.async_copy(pos_v, pos_hbm.at[pl.ds(base, BPW)], sem_i)
    cp_on = pltpu.async_copy(neg_v, neg_hbm.at[pl.ds(base, BPW)], sem_i)
    cp_op.wait()
    cp_on.wait()


@jax.jit
def _bpr(uid, pid, nid, user_table, item_table):
    mesh = plsc.VectorSubcoreMesh(core_axis_name="c", subcore_axis_name="s")
    run = functools.partial(
        pl.kernel,
        out_type=(
            jax.ShapeDtypeStruct((B,), jnp.float32),
            jax.ShapeDtypeStruct((B,), jnp.float32),
        ),
        mesh=mesh,
        scratch_types=(
            pltpu.VMEM((BPW,), jnp.int32),
            pltpu.VMEM((BPW,), jnp.int32),
            pltpu.VMEM((BPW,), jnp.int32),
            pltpu.VMEM((NBUF, C, D), jnp.float32),
            pltpu.VMEM((NBUF, C, D), jnp.float32),
            pltpu.VMEM((NBUF, C, D), jnp.float32),
            pltpu.VMEM((C * L,), jnp.float32),
            pltpu.VMEM((C * L,), jnp.float32),
            pltpu.VMEM((BPW,), jnp.float32),
            pltpu.VMEM((BPW,), jnp.float32),
            pltpu.SemaphoreType.DMA,
            pltpu.SemaphoreType.DMA,
            pltpu.SemaphoreType.DMA,
        ),
    )(_bpr_body)
    return run(uid, pid, nid, user_table, item_table)


def kernel(userId, itemId, neg_itemId, user_table, item_table):
    return _bpr(userId, itemId, neg_itemId, user_table, item_table)
